# speedup vs baseline: 2.4379x; 2.4379x over previous
import functools

import jax
import jax.numpy as jnp
from jax.experimental import pallas as pl

_ANCHORS_W = (116.0, 156.0, 373.0)
_ANCHORS_H = (90.0, 198.0, 326.0)
_G = 76
_NS = _G * _G
_C = 255
_STRIDE = 8.0


def _decode_body(x_ref, o_ref, *, tile_s):
    s = pl.program_id(1)
    xb = x_ref[0]

    sg = jax.nn.sigmoid(xb)
    ex = jnp.exp(xb)

    ch = jax.lax.broadcasted_iota(jnp.int32, (_C, tile_s), 0)
    k = ch % 85
    aw = jnp.where(ch < 85, _ANCHORS_W[0],
                   jnp.where(ch < 170, _ANCHORS_W[1], _ANCHORS_W[2]))
    ah = jnp.where(ch < 85, _ANCHORS_H[0],
                   jnp.where(ch < 170, _ANCHORS_H[1], _ANCHORS_H[2]))
    anch = jnp.where(k == 2, aw, ah).astype(xb.dtype)

    g = s * tile_s + jax.lax.broadcasted_iota(jnp.int32, (_C, tile_s), 1)
    gx = (g % _G).astype(xb.dtype)
    gy = (g // _G).astype(xb.dtype)
    off = jnp.where(k == 0, gx, gy)

    val = jnp.where(k < 2, (sg + off) * _STRIDE,
                    jnp.where(k < 4, ex * anch, sg))
    o_ref[0] = val.T


@jax.jit
def kernel(x):
    b = x.shape[0]
    xr = x.reshape(b, _C, _NS)
    tile_s = 512
    n_tiles = pl.cdiv(_NS, tile_s)
    out = pl.pallas_call(
        functools.partial(_decode_body, tile_s=tile_s),
        grid=(b, n_tiles),
        in_specs=[pl.BlockSpec((1, _C, tile_s), lambda i, j: (i, 0, j))],
        out_specs=pl.BlockSpec((1, tile_s, _C), lambda i, j: (i, j, 0)),
        out_shape=jax.ShapeDtypeStruct((b, _NS, _C), x.dtype),
    )(xr)
    return out.reshape(b, _NS * 3, 85)


# speedup vs baseline: 2.8899x; 1.1854x over previous
import functools

import jax
import jax.numpy as jnp
from jax.experimental import pallas as pl

_ANCHORS_W = (116.0, 156.0, 373.0)
_ANCHORS_H = (90.0, 198.0, 326.0)
_G = 76
_C = 255
_STRIDE = 8.0
_HB = 8
_S = _HB * _G


def _decode_body(x_ref, o_ref):
    j = pl.program_id(1)
    xb = x_ref[0].reshape(_C, _S)

    sg = jax.nn.sigmoid(xb)
    ex = jnp.exp(xb)

    ch = jax.lax.broadcasted_iota(jnp.int32, (_C, _S), 0)
    k = ch % 85
    aw = jnp.where(ch < 85, _ANCHORS_W[0],
                   jnp.where(ch < 170, _ANCHORS_W[1], _ANCHORS_W[2]))
    ah = jnp.where(ch < 85, _ANCHORS_H[0],
                   jnp.where(ch < 170, _ANCHORS_H[1], _ANCHORS_H[2]))
    anch = jnp.where(k == 2, aw, ah).astype(xb.dtype)

    l = jax.lax.broadcasted_iota(jnp.int32, (_C, _S), 1)
    gx = (l % _G).astype(xb.dtype)
    gy = (j * _HB + l // _G).astype(xb.dtype)
    off = jnp.where(k == 0, gx, gy)

    val = jnp.where(k < 2, (sg + off) * _STRIDE,
                    jnp.where(k < 4, ex * anch, sg))
    for a in range(3):
        o_ref[0, pl.Slice(a, _S, 3), :] = val[85 * a:85 * (a + 1), :].T


@jax.jit
def kernel(x):
    b = x.shape[0]
    n_tiles = pl.cdiv(_G, _HB)
    return pl.pallas_call(
        _decode_body,
        grid=(b, n_tiles),
        in_specs=[pl.BlockSpec((1, _C, _HB, _G), lambda i, j: (i, 0, j, 0))],
        out_specs=pl.BlockSpec((1, _S * 3, 85), lambda i, j: (i, j, 0)),
        out_shape=jax.ShapeDtypeStruct((b, _G * _G * 3, 85), x.dtype),
    )(x)


# speedup vs baseline: 5.9176x; 2.0477x over previous
import jax
import jax.numpy as jnp
from jax.experimental import pallas as pl

_ANCHORS_W = (116.0, 156.0, 373.0)
_ANCHORS_H = (90.0, 198.0, 326.0)
_G = 76
_B = 16
_C = 255
_STRIDE = 8.0
_HC = 4
_GC = _HC * _G
_MR = _GC * _B


def _half_consts(base, dtype):
    lane = jax.lax.broadcasted_iota(jnp.int32, (_GC, 128), 1) + base
    k = lane % 85
    aw = jnp.where(lane < 85, _ANCHORS_W[0],
                   jnp.where(lane < 170, _ANCHORS_W[1], _ANCHORS_W[2]))
    ah = jnp.where(lane < 85, _ANCHORS_H[0],
                   jnp.where(lane < 170, _ANCHORS_H[1], _ANCHORS_H[2]))
    anch = jnp.where(k == 2, aw, ah).astype(dtype)
    return k, anch


def _decode_body(x0_ref, x1_ref, o_ref):
    j = pl.program_id(0)
    dt = jnp.float32

    row = jax.lax.broadcasted_iota(jnp.int32, (_GC, 128), 0)
    gx = (row % _G).astype(dt)
    gy = (j * _HC + row // _G).astype(dt)

    k0, anch0 = _half_consts(0, dt)
    k1, anch1 = _half_consts(128, dt)
    off0 = jnp.where(k0 == 0, gx, gy)
    off1 = jnp.where(k1 == 0, gx, gy)

    def decode(u, k, anch, off):
        sg = jax.nn.sigmoid(u)
        ex = jnp.exp(u)
        return jnp.where(k < 2, (sg + off) * _STRIDE,
                         jnp.where(k < 4, ex * anch, sg))

    for b in range(_B):
        rows = pl.Slice(b, _GC, _B)
        u0 = x0_ref[rows, :]
        u1 = x1_ref[rows, :]
        v0 = decode(u0, k0, anch0, off0)
        v1 = decode(u1, k1, anch1, off1)
        o_ref[b, pl.Slice(0, _GC, 3), :] = v0[:, 0:85]
        o_ref[b, pl.Slice(1, _GC, 3), :] = jnp.concatenate(
            [v0[:, 85:128], v1[:, 0:42]], axis=1)
        o_ref[b, pl.Slice(2, _GC, 3), :] = v1[:, 42:127]


@jax.jit
def kernel(x):
    xt = jnp.transpose(x, (2, 3, 0, 1))
    xm = xt.reshape(_G * _G * _B, _C)
    out = pl.pallas_call(
        _decode_body,
        grid=(_G // _HC,),
        in_specs=[pl.BlockSpec((_MR, 128), lambda j: (j, 0)),
                  pl.BlockSpec((_MR, 128), lambda j: (j, 1))],
        out_specs=pl.BlockSpec((_B, 3 * _GC, 85), lambda j: (0, j, 0)),
        out_shape=jax.ShapeDtypeStruct((_B, _G * _G * 3, 85), x.dtype),
    )(xm, xm)
    return out
